# Initial kernel scaffold; baseline (speedup 1.0000x reference)
import jax
import jax.numpy as jnp
from jax.experimental import pallas as pl


def kernel(input, edge_index, edge_vals, W, b):
    raise NotImplementedError("write your pallas kernel here")


# speedup vs baseline: 5.3608x; 5.3608x over previous
import functools

import jax
import jax.numpy as jnp
from jax import lax
from jax.experimental import pallas as pl
from jax.experimental.pallas import tpu as pltpu
from jax.experimental.pallas import tpu_sc as plsc

N_NODES = 10000
N_EDGES = 320000
D = 128

CHUNK = 128
NCHUNKS = N_EDGES // CHUNK
NCORES = 2
NSUB = 16
NWORKERS = NCORES * NSUB
ITERS = -(-NCHUNKS // NWORKERS)
RCHUNK = 200
NRCHUNKS = N_NODES // RCHUNK
RITERS = -(-NRCHUNKS // NSUB)


def _mm_body(x_ref, w_ref, o_ref):
    o_ref[...] = jnp.dot(x_ref[...], w_ref[...],
                         preferred_element_type=jnp.float32)


def _matmul(x, w):
    return pl.pallas_call(
        _mm_body,
        grid=(10,),
        in_specs=[
            pl.BlockSpec((N_NODES // 10, D), lambda i: (i, 0)),
            pl.BlockSpec((D, D), lambda i: (0, 0)),
        ],
        out_specs=pl.BlockSpec((N_NODES // 10, D), lambda i: (i, 0)),
        out_shape=jax.ShapeDtypeStruct((N_NODES, D), jnp.float32),
    )(x, w)


def _comb_body(p_ref, b_ref, o_ref):
    o_ref[...] = p_ref[0] + p_ref[1] + b_ref[...]


def _combine(partials, b):
    return pl.pallas_call(
        _comb_body,
        grid=(10,),
        in_specs=[
            pl.BlockSpec((2, N_NODES // 10, D), lambda i: (0, i, 0)),
            pl.BlockSpec((1, D), lambda i: (0, 0)),
        ],
        out_specs=pl.BlockSpec((N_NODES // 10, D), lambda i: (i, 0)),
        out_shape=jax.ShapeDtypeStruct((N_NODES, D), jnp.float32),
    )(partials, b)


def _spmm(hidden, src, dst, vals):
    mesh = plsc.VectorSubcoreMesh(core_axis_name="core",
                                  subcore_axis_name="subcore")

    @functools.partial(
        pl.kernel,
        out_type=jax.ShapeDtypeStruct((NCORES, N_NODES, D), jnp.float32),
        mesh=mesh,
        scratch_types=[
            pltpu.VMEM((1, CHUNK), jnp.int32),
            pltpu.VMEM((1, CHUNK), jnp.int32),
            pltpu.VMEM((1, CHUNK), jnp.float32),
            pltpu.VMEM((CHUNK, D), jnp.float32),
            pltpu.VMEM((RCHUNK, D), jnp.float32),
            pltpu.VMEM_SHARED((N_NODES, D), jnp.float32),
        ],
    )
    def spmm_kernel(hid_hbm, src_hbm, dst_hbm, val_hbm, part_hbm,
                    sidx_v, didx_v, val_v, rows_v, zeros_v, acc):
        cid = lax.axis_index("core")
        tid = lax.axis_index("subcore")
        wid = tid * NCORES + cid

        @pl.loop(0, RCHUNK)
        def _(r):
            for g in range(D // 16):
                zeros_v[pl.ds(r, 1), pl.ds(g * 16, 16)] = jnp.zeros(
                    (1, 16), jnp.float32)

        for k in range(RITERS):
            rc = k * NSUB + tid

            @pl.when(rc < NRCHUNKS)
            def _():
                pltpu.sync_copy(zeros_v, acc.at[pl.ds(rc * RCHUNK, RCHUNK)])
        plsc.subcore_barrier()

        @pl.loop(0, ITERS)
        def _(i):
            chunk = i * NWORKERS + wid

            @pl.when(chunk < NCHUNKS)
            def _():
                pltpu.sync_copy(src_hbm.at[pl.ds(chunk, 1)], sidx_v)
                pltpu.sync_copy(dst_hbm.at[pl.ds(chunk, 1)], didx_v)
                pltpu.sync_copy(val_hbm.at[pl.ds(chunk, 1)], val_v)
                pltpu.sync_copy(hid_hbm.at[sidx_v.at[0]], rows_v)

                @pl.loop(0, CHUNK // 16)
                def _(eb):
                    vals16 = val_v[pl.ds(0, 1), pl.ds(eb * 16, 16)]
                    for j in range(16):
                        v = vals16[0, j]
                        for g in range(D // 16):
                            sl = (pl.ds(eb * 16 + j, 1), pl.ds(g * 16, 16))
                            rows_v[sl] = rows_v[sl] * v

                pltpu.sync_copy(rows_v, acc.at[didx_v.at[0]], add=True)

        plsc.subcore_barrier()

        for k in range(RITERS):
            rc = k * NSUB + tid

            @pl.when(rc < NRCHUNKS)
            def _():
                pltpu.sync_copy(
                    acc.at[pl.ds(rc * RCHUNK, RCHUNK)],
                    part_hbm.at[cid, pl.ds(rc * RCHUNK, RCHUNK)])

    return spmm_kernel(hidden, src, dst, vals)


def kernel(input, edge_index, edge_vals, W, b):
    hidden = _matmul(input, W)
    dst = edge_index[0].astype(jnp.int32).reshape(NCHUNKS, CHUNK)
    src = edge_index[1].astype(jnp.int32).reshape(NCHUNKS, CHUNK)
    vals = edge_vals.reshape(NCHUNKS, CHUNK)
    partials = _spmm(hidden, src, dst, vals)
    return _combine(partials, b)
